# Initial kernel scaffold; baseline (speedup 1.0000x reference)
import jax
import jax.numpy as jnp
from jax.experimental import pallas as pl


def kernel(x, edge_index, edge_attr, batch, eps1, We1, be1, W1, b1, g1, bt1, m1, v1, eps2, We2, be2, W2, b2, g2, bt2, m2, v2, Wl, bl):
    raise NotImplementedError("write your pallas kernel here")


# speedup vs baseline: 1.2954x; 1.2954x over previous
import functools

import jax
import jax.numpy as jnp
from jax import lax
from jax.experimental import pallas as pl
from jax.experimental.pallas import tpu as pltpu
from jax.experimental.pallas import tpu_sc as plsc

_N = 10000
_E = 160000
_F = 256
_HF = 128
_G = 64
_NSUB = 16
_K = 80
_NP = 10240
_EPT = _E // _NSUB
_NCH = _EPT // _K
_RPT = _NP // _NSUB
_ZR = 128

_LANES = 16
_FV = _HF // _LANES

_GDN = lax.GatherDimensionNumbers(
    offset_dims=(), collapsed_slice_dims=(0,), start_index_map=(0,))


def _edge_body(xt, src, dst, ea, ewb, out, acc, idx_v, dst_v, av_v,
               rows_v, ew_v, buf_v, sem):
    c = lax.axis_index("c")
    s = lax.axis_index("s")

    pltpu.sync_copy(ewb.at[c], ew_v)

    def _zero(i, _):
        for f in range(_FV):
            buf_v[i, pl.ds(f * _LANES, _LANES)] = jnp.zeros((_LANES,), jnp.float32)
        return 0
    lax.fori_loop(0, _ZR, _zero, 0)
    row0 = s * _RPT
    for r in range(_RPT // _ZR):
        pltpu.sync_copy(buf_v, acc.at[pl.ds(row0 + r * _ZR, _ZR)])
    plsc.subcore_barrier()

    ebase = s * _EPT

    def _chunk(i, _):
        base = ebase + i * _K
        pltpu.sync_copy(src.at[pl.ds(base, _K)], idx_v)
        pltpu.sync_copy(dst.at[pl.ds(base, _K)], dst_v)
        pltpu.sync_copy(ea.at[pl.ds(base, _K)], av_v)
        for j in range(_K // _LANES):
            sl = pl.ds(j * _LANES, _LANES)
            idx_v[sl] = idx_v[sl] * 2 + c
        pltpu.async_copy(xt.at[idx_v], rows_v, sem).wait()

        def _group(g, _):
            av16 = av_v[pl.ds(g * _LANES, _LANES)]
            for l in range(_LANES):
                av = lax.gather(
                    av16, jnp.full((_LANES, 1), l, jnp.int32), _GDN, (1,),
                    mode=lax.GatherScatterMode.PROMISE_IN_BOUNDS)
                j = g * _LANES + l
                for f in range(_FV):
                    sl = pl.ds(f * _LANES, _LANES)
                    t = av * ew_v[0, sl] + ew_v[1, sl]
                    rows_v[j, sl] = jnp.maximum(rows_v[j, sl] + t, 0.0)
            return 0
        lax.fori_loop(0, _K // _LANES, _group, 0)

        pltpu.sync_copy(rows_v, acc.at[dst_v], add=True)
        return 0
    lax.fori_loop(0, _NCH, _chunk, 0)

    plsc.subcore_barrier()
    for r in range(_RPT // _ZR):
        sl = pl.ds(row0 + r * _ZR, _ZR)
        pltpu.sync_copy(acc.at[sl], buf_v)
        pltpu.sync_copy(buf_v, out.at[c, sl])


_edge_phase = pl.kernel(
    _edge_body,
    out_type=jax.ShapeDtypeStruct((2, _NP, _HF), jnp.float32),
    mesh=plsc.VectorSubcoreMesh(core_axis_name="c", subcore_axis_name="s"),
    scratch_types=[
        pltpu.VMEM_SHARED((_NP, _HF), jnp.float32),
        pltpu.VMEM((_K,), jnp.int32),
        pltpu.VMEM((_K,), jnp.int32),
        pltpu.VMEM((_K,), jnp.float32),
        pltpu.VMEM((_K, _HF), jnp.float32),
        pltpu.VMEM((2, _HF), jnp.float32),
        pltpu.VMEM((_ZR, _HF), jnp.float32),
        pltpu.SemaphoreType.DMA,
    ],
)


_BLK = 2000
_NBLK = _N // _BLK


def _mlp_body(eps_ref, x_ref, alo_ref, ahi_ref, w_ref, b_ref, o_ref):
    e = eps_ref[0, 0]
    aggr = jnp.concatenate([alo_ref[0], ahi_ref[0]], axis=-1)
    h = x_ref[...] * (1.0 + e) + aggr
    y = jnp.dot(h, w_ref[...], preferred_element_type=jnp.float32,
                precision=lax.Precision.HIGHEST) + b_ref[...]
    o_ref[...] = jnp.where(y >= 0, y, 0.01 * y)


def _mlp(eps, x, aggr, w, b, hdim):
    return pl.pallas_call(
        _mlp_body,
        grid=(_NBLK,),
        in_specs=[
            pl.BlockSpec(memory_space=pltpu.SMEM),
            pl.BlockSpec((_BLK, _F), lambda i: (i, 0)),
            pl.BlockSpec((1, _BLK, _HF), lambda i: (0, i, 0)),
            pl.BlockSpec((1, _BLK, _HF), lambda i: (1, i, 0)),
            pl.BlockSpec((_F, hdim), lambda i: (0, 0)),
            pl.BlockSpec((1, hdim), lambda i: (0, 0)),
        ],
        out_specs=pl.BlockSpec((_BLK, hdim), lambda i: (i, 0)),
        out_shape=jax.ShapeDtypeStruct((_N, hdim), jnp.float32),
    )(eps, x, aggr, aggr, w, b)


def _pool_body(x1_ref, x2_ref, bat_ref, wl_ref, bl_ref, o_ref, p1_acc, p2_acc):
    i = pl.program_id(0)
    bat = bat_ref[0]
    gids = lax.broadcasted_iota(jnp.int32, (_G, _BLK), 0)
    m = jnp.where(gids == bat, 1.0, 0.0)
    pm1 = jnp.dot(m, x1_ref[...], preferred_element_type=jnp.float32,
                  precision=lax.Precision.HIGHEST)
    pm2 = jnp.dot(m, x2_ref[...], preferred_element_type=jnp.float32,
                  precision=lax.Precision.HIGHEST)

    @pl.when(i == 0)
    def _():
        p1_acc[...] = pm1
        p2_acc[...] = pm2

    @pl.when(i > 0)
    def _():
        p1_acc[...] += pm1
        p2_acc[...] += pm2

    @pl.when(i == _NBLK - 1)
    def _():
        p = jnp.concatenate([p1_acc[...], p2_acc[...]], axis=-1)
        z = jnp.dot(p, wl_ref[...], preferred_element_type=jnp.float32,
                    precision=lax.Precision.HIGHEST) + bl_ref[...]
        zmax = jnp.max(z, axis=-1, keepdims=True)
        lse = jnp.log(jnp.sum(jnp.exp(z - zmax), axis=-1, keepdims=True))
        o_ref[...] = z - zmax - lse


def _pool(x1, x2, bat3, wl, bl):
    return pl.pallas_call(
        _pool_body,
        grid=(_NBLK,),
        in_specs=[
            pl.BlockSpec((_BLK, _F), lambda i: (i, 0)),
            pl.BlockSpec((_BLK, _HF), lambda i: (i, 0)),
            pl.BlockSpec((1, 1, _BLK), lambda i: (i, 0, 0)),
            pl.BlockSpec((_F + _HF, 10), lambda i: (0, 0)),
            pl.BlockSpec((1, 10), lambda i: (0, 0)),
        ],
        out_specs=pl.BlockSpec((_G, 10), lambda i: (0, 0)),
        out_shape=jax.ShapeDtypeStruct((_G, 10), jnp.float32),
        scratch_shapes=[
            pltpu.VMEM((_G, _F), jnp.float32),
            pltpu.VMEM((_G, _HF), jnp.float32),
        ],
    )(x1, x2, bat3, wl, bl)


def _fold_bn(W, b, g, bt, m, v):
    scale = g / jnp.sqrt(v + 1e-5)
    return W * scale[None, :], (b - m) * scale + bt


def kernel(x, edge_index, edge_attr, batch, eps1, We1, be1, W1, b1, g1, bt1,
           m1, v1, eps2, We2, be2, W2, b2, g2, bt2, m2, v2, Wl, bl):
    src = edge_index[0]
    dst = edge_index[1]
    ea = edge_attr[:, 0]
    ewb1 = jnp.stack([We1[0].reshape(2, _HF), be1.reshape(2, _HF)], axis=1)
    ewb2 = jnp.stack([We2[0].reshape(2, _HF), be2.reshape(2, _HF)], axis=1)
    W1p, b1p = _fold_bn(W1, b1, g1, bt1, m1, v1)
    W2p, b2p = _fold_bn(W2, b2, g2, bt2, m2, v2)
    bat3 = batch.reshape(_NBLK, 1, _BLK)

    aggr1 = _edge_phase(x.reshape(2 * _N, _HF), src, dst, ea, ewb1)
    x1 = _mlp(eps1.reshape(1, 1), x, aggr1, W1p, b1p.reshape(1, -1), _F)
    aggr2 = _edge_phase(x1.reshape(2 * _N, _HF), src, dst, ea, ewb2)
    x2 = _mlp(eps2.reshape(1, 1), x1, aggr2, W2p, b2p.reshape(1, -1), _HF)
    return _pool(x1, x2, bat3, Wl, bl.reshape(1, -1))


# speedup vs baseline: 4.8574x; 3.7497x over previous
import functools

import jax
import jax.numpy as jnp
from jax import lax
from jax.experimental import pallas as pl
from jax.experimental.pallas import tpu as pltpu
from jax.experimental.pallas import tpu_sc as plsc

_N = 10000
_E = 160000
_F = 256
_HF = 128
_G = 64
_NSUB = 16
_K = 80
_NP = 10240
_EPT = _E // _NSUB
_NCH = _EPT // _K
_RPT = _NP // _NSUB
_ZR = 128

_LANES = 16
_FV = _HF // _LANES

_GDN = lax.GatherDimensionNumbers(
    offset_dims=(), collapsed_slice_dims=(0,), start_index_map=(0,))


def _edge_body(xt, ed, ea, ewb, out, acc, ebuf, abuf, rows, ew_v, gsems, ssems):
    c = lax.axis_index("c")
    s = lax.axis_index("s")

    pltpu.sync_copy(ewb.at[c], ew_v)

    def _zero(i, _):
        for f in range(_FV):
            rows[0][i, pl.ds(f * _LANES, _LANES)] = jnp.zeros((_LANES,), jnp.float32)
        return 0
    lax.fori_loop(0, _K, _zero, 0)
    row0 = s * _RPT
    for r in range(_RPT // _K):
        pltpu.sync_copy(rows[0], acc.at[pl.ds(row0 + r * _K, _K)])
    plsc.subcore_barrier()

    ew0 = [ew_v[0, pl.ds(f * _LANES, _LANES)] for f in range(_FV)]
    ew1 = [ew_v[1, pl.ds(f * _LANES, _LANES)] for f in range(_FV)]

    def _e_fetch(i, b):
        pltpu.sync_copy(ed.at[s, i], ebuf.at[b])
        pltpu.sync_copy(ea.at[s, i], abuf.at[b])
        for j in range(_K // _LANES):
            sl = pl.ds(j * _LANES, _LANES)
            ebuf[b, 0, sl] = ebuf[b, 0, sl] * 2 + c

    def _g_start(i, b):
        pltpu.async_copy(xt.at[ebuf.at[b, 0]], rows[b], gsems[b])

    def _g_wait(b):
        pltpu.make_async_copy(xt.at[ebuf.at[b, 0]], rows[b], gsems[b]).wait()

    def _s_start(b):
        pltpu.async_copy(rows[b], acc.at[ebuf.at[b, 1]], ssems[b], add=True)

    def _s_wait(b):
        pltpu.make_async_copy(rows[b], acc.at[ebuf.at[b, 1]], ssems[b]).wait()

    def _compute(b):
        rb = rows[b]

        def _group(g, _):
            av16 = abuf[b, 0, pl.ds(g * _LANES, _LANES)]

            def _lane(l, _):
                av = lax.gather(
                    av16, lax.broadcast_in_dim(l, (_LANES, 1), ()), _GDN, (1,),
                    mode=lax.GatherScatterMode.PROMISE_IN_BOUNDS)
                j = g * _LANES + l
                for f in range(_FV):
                    sl = pl.ds(f * _LANES, _LANES)
                    t = av * ew0[f] + ew1[f]
                    rb[j, sl] = jnp.maximum(rb[j, sl] + t, 0.0)
                return 0
            lax.fori_loop(0, _LANES, _lane, 0)
            return 0
        lax.fori_loop(0, _K // _LANES, _group, 0)

    _e_fetch(0, 0)
    _g_start(0, 0)
    _e_fetch(1, 1)
    _g_start(1, 1)
    _g_wait(0)
    _compute(0)
    _s_start(0)
    _e_fetch(2, 2)
    _g_start(2, 2)
    _g_wait(1)
    _compute(1)
    _s_start(1)

    def _steady(i, b):
        bn = (b + 1) % 3
        _s_wait(bn)
        _e_fetch(i + 1, bn)
        _g_start(i + 1, bn)
        _g_wait(b)
        _compute(b)
        _s_start(b)

    def _trip(p, _):
        i = 3 * p + 2
        _steady(i, 2)
        _steady(i + 1, 0)
        _steady(i + 2, 1)
        return 0
    lax.fori_loop(0, (_NCH - 5) // 3, _trip, 0)

    _steady(_NCH - 3, 2)
    _steady(_NCH - 2, 0)
    _g_wait(1)
    _compute(1)
    _s_start(1)
    _s_wait(2)
    _s_wait(0)
    _s_wait(1)

    plsc.subcore_barrier()
    for r in range(_RPT // _K):
        sl = pl.ds(row0 + r * _K, _K)
        pltpu.sync_copy(acc.at[sl], rows[0])
        pltpu.sync_copy(rows[0], out.at[c, sl])


_edge_phase = pl.kernel(
    _edge_body,
    out_type=jax.ShapeDtypeStruct((2, _NP, _HF), jnp.float32),
    mesh=plsc.VectorSubcoreMesh(core_axis_name="c", subcore_axis_name="s"),
    scratch_types=[
        pltpu.VMEM_SHARED((_NP, _HF), jnp.float32),
        pltpu.VMEM((3, 2, _K), jnp.int32),
        pltpu.VMEM((3, 1, _K), jnp.float32),
        [pltpu.VMEM((_K, _HF), jnp.float32)] * 3,
        pltpu.VMEM((2, _HF), jnp.float32),
        [pltpu.SemaphoreType.DMA] * 3,
        [pltpu.SemaphoreType.DMA] * 3,
    ],
)


_BLK = 2000
_NBLK = _N // _BLK


def _mlp_body(eps_ref, x_ref, alo_ref, ahi_ref, w_ref, b_ref, o_ref):
    e = eps_ref[0, 0]
    aggr = jnp.concatenate([alo_ref[0], ahi_ref[0]], axis=-1)
    h = x_ref[...] * (1.0 + e) + aggr
    y = jnp.dot(h, w_ref[...], preferred_element_type=jnp.float32,
                precision=lax.Precision.HIGHEST) + b_ref[...]
    o_ref[...] = jnp.where(y >= 0, y, 0.01 * y)


def _mlp(eps, x, aggr, w, b, hdim):
    return pl.pallas_call(
        _mlp_body,
        grid=(_NBLK,),
        in_specs=[
            pl.BlockSpec(memory_space=pltpu.SMEM),
            pl.BlockSpec((_BLK, _F), lambda i: (i, 0)),
            pl.BlockSpec((1, _BLK, _HF), lambda i: (0, i, 0)),
            pl.BlockSpec((1, _BLK, _HF), lambda i: (1, i, 0)),
            pl.BlockSpec((_F, hdim), lambda i: (0, 0)),
            pl.BlockSpec((1, hdim), lambda i: (0, 0)),
        ],
        out_specs=pl.BlockSpec((_BLK, hdim), lambda i: (i, 0)),
        out_shape=jax.ShapeDtypeStruct((_N, hdim), jnp.float32),
    )(eps, x, aggr, aggr, w, b)


def _pool_body(x1_ref, x2_ref, bat_ref, wl_ref, bl_ref, o_ref, p1_acc, p2_acc):
    i = pl.program_id(0)
    bat = bat_ref[0]
    gids = lax.broadcasted_iota(jnp.int32, (_G, _BLK), 0)
    m = jnp.where(gids == bat, 1.0, 0.0)
    pm1 = jnp.dot(m, x1_ref[...], preferred_element_type=jnp.float32,
                  precision=lax.Precision.HIGHEST)
    pm2 = jnp.dot(m, x2_ref[...], preferred_element_type=jnp.float32,
                  precision=lax.Precision.HIGHEST)

    @pl.when(i == 0)
    def _():
        p1_acc[...] = pm1
        p2_acc[...] = pm2

    @pl.when(i > 0)
    def _():
        p1_acc[...] += pm1
        p2_acc[...] += pm2

    @pl.when(i == _NBLK - 1)
    def _():
        p = jnp.concatenate([p1_acc[...], p2_acc[...]], axis=-1)
        z = jnp.dot(p, wl_ref[...], preferred_element_type=jnp.float32,
                    precision=lax.Precision.HIGHEST) + bl_ref[...]
        zmax = jnp.max(z, axis=-1, keepdims=True)
        lse = jnp.log(jnp.sum(jnp.exp(z - zmax), axis=-1, keepdims=True))
        o_ref[...] = z - zmax - lse


def _pool(x1, x2, bat3, wl, bl):
    return pl.pallas_call(
        _pool_body,
        grid=(_NBLK,),
        in_specs=[
            pl.BlockSpec((_BLK, _F), lambda i: (i, 0)),
            pl.BlockSpec((_BLK, _HF), lambda i: (i, 0)),
            pl.BlockSpec((1, 1, _BLK), lambda i: (i, 0, 0)),
            pl.BlockSpec((_F + _HF, 10), lambda i: (0, 0)),
            pl.BlockSpec((1, 10), lambda i: (0, 0)),
        ],
        out_specs=pl.BlockSpec((_G, 10), lambda i: (0, 0)),
        out_shape=jax.ShapeDtypeStruct((_G, 10), jnp.float32),
        scratch_shapes=[
            pltpu.VMEM((_G, _F), jnp.float32),
            pltpu.VMEM((_G, _HF), jnp.float32),
        ],
    )(x1, x2, bat3, wl, bl)


def _fold_bn(W, b, g, bt, m, v):
    scale = g / jnp.sqrt(v + 1e-5)
    return W * scale[None, :], (b - m) * scale + bt


def kernel(x, edge_index, edge_attr, batch, eps1, We1, be1, W1, b1, g1, bt1,
           m1, v1, eps2, We2, be2, W2, b2, g2, bt2, m2, v2, Wl, bl):
    src = edge_index[0]
    dst = edge_index[1]
    ea = edge_attr[:, 0]
    ewb1 = jnp.stack([We1[0].reshape(2, _HF), be1.reshape(2, _HF)], axis=1)
    ewb2 = jnp.stack([We2[0].reshape(2, _HF), be2.reshape(2, _HF)], axis=1)
    W1p, b1p = _fold_bn(W1, b1, g1, bt1, m1, v1)
    W2p, b2p = _fold_bn(W2, b2, g2, bt2, m2, v2)
    bat3 = batch.reshape(_NBLK, 1, _BLK)

    ed = jnp.stack(
        [src.reshape(_NSUB, _NCH, _K), dst.reshape(_NSUB, _NCH, _K)], axis=2)
    ea3 = ea.reshape(_NSUB, _NCH, 1, _K)

    aggr1 = _edge_phase(x.reshape(2 * _N, _HF), ed, ea3, ewb1)
    x1 = _mlp(eps1.reshape(1, 1), x, aggr1, W1p, b1p.reshape(1, -1), _F)
    aggr2 = _edge_phase(x1.reshape(2 * _N, _HF), ed, ea3, ewb2)
    x2 = _mlp(eps2.reshape(1, 1), x1, aggr2, W2p, b2p.reshape(1, -1), _HF)
    return _pool(x1, x2, bat3, Wl, bl.reshape(1, -1))


# speedup vs baseline: 7.0335x; 1.4480x over previous
import functools

import jax
import jax.numpy as jnp
from jax import lax
from jax.experimental import pallas as pl
from jax.experimental.pallas import tpu as pltpu
from jax.experimental.pallas import tpu_sc as plsc

_N = 10000
_E = 160000
_F = 256
_HF = 128
_G = 64
_NSUB = 16
_K = 80
_NP = 10240
_EPT = _E // _NSUB
_NCH = _EPT // _K
_RPT = _NP // _NSUB
_ZR = 128

_LANES = 16
_FV = _HF // _LANES

_GDN = lax.GatherDimensionNumbers(
    offset_dims=(), collapsed_slice_dims=(0,), start_index_map=(0,))


def _edge_body(xt, ed, ea, ewb, out, acc, ebuf, abuf, rows, ew_v,
               gsems, ssems, esems, fsems):
    c = lax.axis_index("c")
    s = lax.axis_index("s")

    pltpu.sync_copy(ewb.at[c], ew_v)

    def _zero(i, _):
        for f in range(_FV):
            rows[0][i, pl.ds(f * _LANES, _LANES)] = jnp.zeros((_LANES,), jnp.float32)
        return 0
    lax.fori_loop(0, _K, _zero, 0)
    row0 = s * _RPT
    for r in range(_RPT // _K):
        pltpu.sync_copy(rows[0], acc.at[pl.ds(row0 + r * _K, _K)])
    plsc.subcore_barrier()

    ew0 = [ew_v[0, pl.ds(f * _LANES, _LANES)] for f in range(_FV)]

    def _e_start(i, e):
        pltpu.async_copy(ed.at[s, i], ebuf.at[e], esems[e])
        pltpu.async_copy(ea.at[s, i], abuf.at[e], fsems[e])

    def _e_wait(i, e):
        pltpu.make_async_copy(ed.at[s, i], ebuf.at[e], esems[e]).wait()
        pltpu.make_async_copy(ea.at[s, i], abuf.at[e], fsems[e]).wait()
        for j in range(_K // _LANES):
            sl = pl.ds(j * _LANES, _LANES)
            ebuf[e, 0, sl] = ebuf[e, 0, sl] * 2 + c

    def _g_start(i, b, e):
        pltpu.async_copy(xt.at[ebuf.at[e, 0]], rows[b], gsems[b])

    def _g_wait(b, e):
        pltpu.make_async_copy(xt.at[ebuf.at[e, 0]], rows[b], gsems[b]).wait()

    def _s_start(b, e):
        pltpu.async_copy(rows[b], acc.at[ebuf.at[e, 1]], ssems[b], add=True)

    def _s_wait(b, e):
        pltpu.make_async_copy(rows[b], acc.at[ebuf.at[e, 1]], ssems[b]).wait()

    def _compute(b, e):
        rb = rows[b]

        def _group(g, _):
            av16 = abuf[e, 0, pl.ds(g * _LANES, _LANES)]

            def _lane(l, _):
                for u in range(2):
                    lu = l * 2 + u
                    av = lax.gather(
                        av16, lax.broadcast_in_dim(lu, (_LANES, 1), ()),
                        _GDN, (1,),
                        mode=lax.GatherScatterMode.PROMISE_IN_BOUNDS)
                    j = g * _LANES + lu
                    for f in range(_FV):
                        sl = pl.ds(f * _LANES, _LANES)
                        rb[j, sl] = jnp.maximum(rb[j, sl] + av * ew0[f], 0.0)
                return 0
            lax.fori_loop(0, _LANES // 2, _lane, 0)
            return 0
        lax.fori_loop(0, _K // _LANES, _group, 0)

    def _chunk(i, b, e, wait_sc=True, fetch2=True, gnext=True):
        bn = (b + 1) % 3
        if wait_sc:
            _s_wait(bn, (e + 4) % 6)
        if fetch2:
            _e_start(i + 2, (e + 2) % 6)
        if gnext:
            _e_wait(i + 1, (e + 1) % 6)
            _g_start(i + 1, bn, (e + 1) % 6)
        _g_wait(b, e)
        _compute(b, e)
        _s_start(b, e)

    _e_start(0, 0)
    _e_start(1, 1)
    _e_wait(0, 0)
    _g_start(0, 0, 0)
    _chunk(0, 0, 0, wait_sc=False)
    _chunk(1, 1, 1, wait_sc=False)

    def _trip(p, _):
        i = 6 * p + 2
        _chunk(i, 2, 2)
        _chunk(i + 1, 0, 3)
        _chunk(i + 2, 1, 4)
        _chunk(i + 3, 2, 5)
        _chunk(i + 4, 0, 0)
        _chunk(i + 5, 1, 1)
        return 0
    lax.fori_loop(0, (_NCH - 5) // 6, _trip, 0)

    _chunk(_NCH - 3, 2, 2)
    _chunk(_NCH - 2, 0, 3, fetch2=False)
    _chunk(_NCH - 1, 1, 4, fetch2=False, gnext=False)
    _s_wait(0, 3)
    _s_wait(1, 4)

    plsc.subcore_barrier()
    for r in range(_RPT // _K):
        sl = pl.ds(row0 + r * _K, _K)
        pltpu.sync_copy(acc.at[sl], rows[0])
        pltpu.sync_copy(rows[0], out.at[c, sl])


_edge_phase = pl.kernel(
    _edge_body,
    out_type=jax.ShapeDtypeStruct((2, _NP, _HF), jnp.float32),
    mesh=plsc.VectorSubcoreMesh(core_axis_name="c", subcore_axis_name="s"),
    scratch_types=[
        pltpu.VMEM_SHARED((_NP, _HF), jnp.float32),
        pltpu.VMEM((6, 2, _K), jnp.int32),
        pltpu.VMEM((6, 1, _K), jnp.float32),
        [pltpu.VMEM((_K, _HF), jnp.float32)] * 3,
        pltpu.VMEM((1, _HF), jnp.float32),
        [pltpu.SemaphoreType.DMA] * 3,
        [pltpu.SemaphoreType.DMA] * 3,
        [pltpu.SemaphoreType.DMA] * 6,
        [pltpu.SemaphoreType.DMA] * 6,
    ],
)


_BLK = 2000
_NBLK = _N // _BLK


def _mlp_body(eps_ref, x_ref, alo_ref, ahi_ref, w_ref, b_ref, *rest):
    e = eps_ref[0, 0]
    aggr = jnp.concatenate([alo_ref[0], ahi_ref[0]], axis=-1)
    h = x_ref[...] * (1.0 + e) + aggr
    y = jnp.dot(h, w_ref[...], preferred_element_type=jnp.float32,
                precision=lax.Precision.HIGHEST) + b_ref[...]
    y = jnp.where(y >= 0, y, 0.01 * y)
    if len(rest) == 1:
        rest[0][...] = y
    else:
        bn_ref, o_ref, o2_ref = rest
        o_ref[...] = y
        o2_ref[...] = y + bn_ref[...]


def _mlp(eps, x, aggr, w, b, hdim, bnext=None):
    out = pl.pallas_call(
        _mlp_body,
        grid=(_NBLK,),
        in_specs=[
            pl.BlockSpec(memory_space=pltpu.SMEM),
            pl.BlockSpec((_BLK, _F), lambda i: (i, 0)),
            pl.BlockSpec((1, _BLK, _HF), lambda i: (0, i, 0)),
            pl.BlockSpec((1, _BLK, _HF), lambda i: (1, i, 0)),
            pl.BlockSpec((_F, hdim), lambda i: (0, 0)),
            pl.BlockSpec((1, hdim), lambda i: (0, 0)),
        ] + ([pl.BlockSpec((1, hdim), lambda i: (0, 0))] if bnext is not None
             else []),
        out_specs=[pl.BlockSpec((_BLK, hdim), lambda i: (i, 0))] * (
            2 if bnext is not None else 1),
        out_shape=[jax.ShapeDtypeStruct((_N, hdim), jnp.float32)] * (
            2 if bnext is not None else 1),
    )(eps, x, aggr, aggr, w, b,
      *([bnext] if bnext is not None else []))
    return out if bnext is not None else out[0]


def _biasadd_body(x_ref, b_ref, o_ref):
    o_ref[...] = x_ref[...] + b_ref[...]


def _biasadd(x, b):
    return pl.pallas_call(
        _biasadd_body,
        grid=(_NBLK,),
        in_specs=[
            pl.BlockSpec((_BLK, _F), lambda i: (i, 0)),
            pl.BlockSpec((1, _F), lambda i: (0, 0)),
        ],
        out_specs=pl.BlockSpec((_BLK, _F), lambda i: (i, 0)),
        out_shape=jax.ShapeDtypeStruct((_N, _F), jnp.float32),
    )(x, b)


def _pool_body(x1_ref, x2_ref, bat_ref, wl_ref, bl_ref, o_ref, p1_acc, p2_acc):
    i = pl.program_id(0)
    bat = bat_ref[0]
    gids = lax.broadcasted_iota(jnp.int32, (_G, _BLK), 0)
    m = jnp.where(gids == bat, 1.0, 0.0)
    pm1 = jnp.dot(m, x1_ref[...], preferred_element_type=jnp.float32,
                  precision=lax.Precision.HIGHEST)
    pm2 = jnp.dot(m, x2_ref[...], preferred_element_type=jnp.float32,
                  precision=lax.Precision.HIGHEST)

    @pl.when(i == 0)
    def _():
        p1_acc[...] = pm1
        p2_acc[...] = pm2

    @pl.when(i > 0)
    def _():
        p1_acc[...] += pm1
        p2_acc[...] += pm2

    @pl.when(i == _NBLK - 1)
    def _():
        p = jnp.concatenate([p1_acc[...], p2_acc[...]], axis=-1)
        z = jnp.dot(p, wl_ref[...], preferred_element_type=jnp.float32,
                    precision=lax.Precision.HIGHEST) + bl_ref[...]
        zmax = jnp.max(z, axis=-1, keepdims=True)
        lse = jnp.log(jnp.sum(jnp.exp(z - zmax), axis=-1, keepdims=True))
        o_ref[...] = z - zmax - lse


def _pool(x1, x2, bat3, wl, bl):
    return pl.pallas_call(
        _pool_body,
        grid=(_NBLK,),
        in_specs=[
            pl.BlockSpec((_BLK, _F), lambda i: (i, 0)),
            pl.BlockSpec((_BLK, _HF), lambda i: (i, 0)),
            pl.BlockSpec((1, 1, _BLK), lambda i: (i, 0, 0)),
            pl.BlockSpec((_F + _HF, 10), lambda i: (0, 0)),
            pl.BlockSpec((1, 10), lambda i: (0, 0)),
        ],
        out_specs=pl.BlockSpec((_G, 10), lambda i: (0, 0)),
        out_shape=jax.ShapeDtypeStruct((_G, 10), jnp.float32),
        scratch_shapes=[
            pltpu.VMEM((_G, _F), jnp.float32),
            pltpu.VMEM((_G, _HF), jnp.float32),
        ],
    )(x1, x2, bat3, wl, bl)


def _fold_bn(W, b, g, bt, m, v):
    scale = g / jnp.sqrt(v + 1e-5)
    return W * scale[None, :], (b - m) * scale + bt


def kernel(x, edge_index, edge_attr, batch, eps1, We1, be1, W1, b1, g1, bt1,
           m1, v1, eps2, We2, be2, W2, b2, g2, bt2, m2, v2, Wl, bl):
    src = edge_index[0]
    dst = edge_index[1]
    ea = edge_attr[:, 0]
    ewb1 = We1[0].reshape(2, 1, _HF)
    ewb2 = We2[0].reshape(2, 1, _HF)
    W1p, b1p = _fold_bn(W1, b1, g1, bt1, m1, v1)
    W2p, b2p = _fold_bn(W2, b2, g2, bt2, m2, v2)
    bat3 = batch.reshape(_NBLK, 1, _BLK)

    ed = jnp.stack(
        [src.reshape(_NSUB, _NCH, _K), dst.reshape(_NSUB, _NCH, _K)], axis=2)
    ea3 = ea.reshape(_NSUB, _NCH, 1, _K)

    xb = _biasadd(x, be1.reshape(1, -1))
    aggr1 = _edge_phase(xb.reshape(2 * _N, _HF), ed, ea3, ewb1)
    x1, x1b = _mlp(eps1.reshape(1, 1), x, aggr1, W1p, b1p.reshape(1, -1), _F,
                   bnext=be2.reshape(1, -1))
    aggr2 = _edge_phase(x1b.reshape(2 * _N, _HF), ed, ea3, ewb2)
    x2 = _mlp(eps2.reshape(1, 1), x1, aggr2, W2p, b2p.reshape(1, -1), _HF)
    return _pool(x1, x2, bat3, Wl, bl.reshape(1, -1))
